# baseline (speedup 1.0000x reference)
import functools

import jax
import jax.numpy as jnp
from jax import lax
from jax.experimental import pallas as pl
from jax.experimental.pallas import tpu as pltpu
from jax.experimental.pallas import tpu_sc as plsc

_B = 16384
_K = 8192
_D = 32
_TBL = 256
_RC = 32
_NC = _K // _RC
_NCH = _NC // 2


def _codes_body(z2_ref, e2_ref, zblk_ref, cb2_ref, codes_ref):
    z2v = z2_ref[...]
    zbt = zblk_ref[...].T.astype(jnp.bfloat16)
    iota_r = lax.broadcasted_iota(jnp.int32, (_RC, _TBL), 0)

    def chunk_dot(c):
        cb_c = cb2_ref[pl.ds(c * _RC, _RC), :]
        return lax.dot_general(cb_c, zbt, (((1,), (0,)), ((), ())),
                               preferred_element_type=jnp.float32)

    def chunk_dist(c, ze2c):
        e2c = e2_ref[pl.ds(c * _RC, _RC), :]
        return (z2v + e2c) - ze2c

    def window(c_lo):
        def upd(c, v, M, C):
            msk = v < M
            return jnp.where(msk, v, M), jnp.where(msk, c, C)

        M = jnp.full((_RC, _TBL), jnp.inf, jnp.float32)
        C = jnp.zeros((_RC, _TBL), jnp.int32)
        for i in range(_NCH):
            c = c_lo + i
            M, C = upd(c, chunk_dist(c, chunk_dot(c)), M, C)

        Kg = _RC * C + iota_r
        rc = _RC
        while rc > 1:
            h = rc // 2
            Ma, Mb, Ka, Kb = M[:h], M[h:rc], Kg[:h], Kg[h:rc]
            t = (Ma < Mb) | ((Ma == Mb) & (Ka < Kb))
            M = jnp.where(t, Ma, Mb)
            Kg = jnp.where(t, Ka, Kb)
            rc = h
        return M, Kg

    m0, k0 = window(0)
    m1, k1 = window(_NCH)
    a = m0.astype(jnp.bfloat16).astype(jnp.float32)
    codes_ref[...] = jnp.where(a <= m1, k0, k1).reshape(_TBL)


def _compute_codes(z2r, e2c, z_e, cb2):
    return pl.pallas_call(
        _codes_body,
        grid=(_B // _TBL,),
        in_specs=[
            pl.BlockSpec((1, _TBL), lambda i: (0, i)),
            pl.BlockSpec((_K, 1), lambda i: (0, 0)),
            pl.BlockSpec((_TBL, _D), lambda i: (i, 0)),
            pl.BlockSpec((_K, _D), lambda i: (0, 0)),
        ],
        out_specs=pl.BlockSpec((_TBL,), lambda i: (i,)),
        out_shape=jax.ShapeDtypeStruct((_B,), jnp.int32),
    )(z2r, e2c, z_e, cb2)


_NW = 32
_BPW = _B // _NW
_CH = 128


def _gather_body(table_hbm, idx_hbm, out_hbm, idx_v, rows_v, sem):
    wid = lax.axis_index("s") * 2 + lax.axis_index("c")
    base = wid * _BPW
    pltpu.sync_copy(idx_hbm.at[pl.ds(base, _BPW)], idx_v)
    copies = [
        pltpu.async_copy(
            table_hbm.at[idx_v.at[pl.ds(j * _CH, _CH)]],
            rows_v.at[pl.ds(j * _CH, _CH)],
            sem,
        )
        for j in range(_BPW // _CH)
    ]
    for c in copies:
        c.wait()
    pltpu.sync_copy(rows_v, out_hbm.at[pl.ds(base, _BPW)])


@functools.cache
def _gather_rows():
    return pl.kernel(
        _gather_body,
        out_type=jax.ShapeDtypeStruct((_B, _D), jnp.float32),
        mesh=plsc.VectorSubcoreMesh(core_axis_name="c", subcore_axis_name="s"),
        scratch_types=[
            pltpu.VMEM((_BPW,), jnp.int32),
            pltpu.VMEM((_BPW, _D), jnp.float32),
            pltpu.SemaphoreType.DMA,
        ],
        compiler_params=pltpu.CompilerParams(use_tc_tiling_on_sc=False),
    )


def kernel(z_e, codebook):
    z2r = jnp.sum(z_e ** 2, axis=1)[None, :]
    e2c = jnp.sum(codebook ** 2, axis=1)[:, None]
    cb2 = (2.0 * codebook).astype(jnp.bfloat16)
    codes = _compute_codes(z2r, e2c, z_e, cb2)
    z_q = _gather_rows()(codebook, codes)
    return (z_q, codes)


# speedup vs baseline: 1.0193x; 1.0193x over previous
import functools

import jax
import jax.numpy as jnp
from jax import lax
from jax.experimental import pallas as pl
from jax.experimental.pallas import tpu as pltpu
from jax.experimental.pallas import tpu_sc as plsc

_B = 16384
_K = 8192
_D = 32
_TBL = 256
_RC = 64
_NC = _K // _RC
_NCH = _NC // 2


def _codes_body(z2_ref, e2_ref, zbt_ref, cb2_ref, codes_ref):
    z2v = z2_ref[...]
    zbt = zbt_ref[...]
    iota_r = lax.broadcasted_iota(jnp.int32, (_RC, _TBL), 0)

    def chunk_dot(c):
        cb_c = cb2_ref[pl.ds(c * _RC, _RC), :]
        return lax.dot_general(cb_c, zbt, (((1,), (0,)), ((), ())),
                               preferred_element_type=jnp.float32)

    def chunk_dist(c, ze2c):
        e2c = e2_ref[pl.ds(c * _RC, _RC), :]
        return (z2v + e2c) - ze2c

    def window(c_lo):
        def upd(c, v, M, C):
            msk = v < M
            return jnp.where(msk, v, M), jnp.where(msk, c, C)

        M = jnp.full((_RC, _TBL), jnp.inf, jnp.float32)
        C = jnp.zeros((_RC, _TBL), jnp.int32)
        for i in range(_NCH):
            c = c_lo + i
            M, C = upd(c, chunk_dist(c, chunk_dot(c)), M, C)

        Kg = _RC * C + iota_r
        rc = _RC
        while rc > 1:
            h = rc // 2
            Ma, Mb, Ka, Kb = M[:h], M[h:rc], Kg[:h], Kg[h:rc]
            t = (Ma < Mb) | ((Ma == Mb) & (Ka < Kb))
            M = jnp.where(t, Ma, Mb)
            Kg = jnp.where(t, Ka, Kb)
            rc = h
        return M, Kg

    m0, k0 = window(0)
    m1, k1 = window(_NCH)
    a = m0.astype(jnp.bfloat16).astype(jnp.float32)
    codes_ref[...] = jnp.where(a <= m1, k0, k1).reshape(_TBL)


def _compute_codes(z2r, e2c, zbt, cb2):
    return pl.pallas_call(
        _codes_body,
        grid=(_B // _TBL,),
        in_specs=[
            pl.BlockSpec((1, _TBL), lambda i: (0, i)),
            pl.BlockSpec((_K, 1), lambda i: (0, 0)),
            pl.BlockSpec((_D, _TBL), lambda i: (0, i)),
            pl.BlockSpec((_K, _D), lambda i: (0, 0)),
        ],
        out_specs=pl.BlockSpec((_TBL,), lambda i: (i,)),
        out_shape=jax.ShapeDtypeStruct((_B,), jnp.int32),
    )(z2r, e2c, zbt, cb2)


_NW = 32
_BPW = _B // _NW
_CH = 128


def _gather_body(table_hbm, idx_hbm, out_hbm, idx_v, rows_v, sem):
    wid = lax.axis_index("s") * 2 + lax.axis_index("c")
    base = wid * _BPW
    pltpu.sync_copy(idx_hbm.at[pl.ds(base, _BPW)], idx_v)
    copies = [
        pltpu.async_copy(
            table_hbm.at[idx_v.at[pl.ds(j * _CH, _CH)]],
            rows_v.at[pl.ds(j * _CH, _CH)],
            sem,
        )
        for j in range(_BPW // _CH)
    ]
    for c in copies:
        c.wait()
    pltpu.sync_copy(rows_v, out_hbm.at[pl.ds(base, _BPW)])


@functools.cache
def _gather_rows():
    return pl.kernel(
        _gather_body,
        out_type=jax.ShapeDtypeStruct((_B, _D), jnp.float32),
        mesh=plsc.VectorSubcoreMesh(core_axis_name="c", subcore_axis_name="s"),
        scratch_types=[
            pltpu.VMEM((_BPW,), jnp.int32),
            pltpu.VMEM((_BPW, _D), jnp.float32),
            pltpu.SemaphoreType.DMA,
        ],
        compiler_params=pltpu.CompilerParams(use_tc_tiling_on_sc=False),
    )


def kernel(z_e, codebook):
    z2r = jnp.sum(z_e ** 2, axis=1)[None, :]
    e2c = jnp.sum(codebook ** 2, axis=1, keepdims=True)
    zbt = z_e.T.astype(jnp.bfloat16)
    cb2 = (2.0 * codebook).astype(jnp.bfloat16)
    codes = _compute_codes(z2r, e2c, zbt, cb2)
    z_q = _gather_rows()(codebook, codes)
    return (z_q, codes)


# speedup vs baseline: 1.0694x; 1.0492x over previous
import functools

import jax
import jax.numpy as jnp
from jax import lax
from jax.experimental import pallas as pl
from jax.experimental.pallas import tpu as pltpu
from jax.experimental.pallas import tpu_sc as plsc

_B = 16384
_K = 8192
_D = 32
_TBL = 256
_RC = 32
_NC = _K // _RC
_NCH = _NC // 2


def _codes_body(z2_ref, e2_ref, zbt_ref, cb2_ref, codes_ref):
    z2v = z2_ref[...]
    zbt = zbt_ref[...]
    iota_r = lax.broadcasted_iota(jnp.int32, (_RC, _TBL), 0)

    def chunk_dot(c):
        cb_c = cb2_ref[pl.ds(c * _RC, _RC), :]
        return lax.dot_general(cb_c, zbt, (((1,), (0,)), ((), ())),
                               preferred_element_type=jnp.float32)

    def chunk_dist(c, ze2c):
        e2c = e2_ref[pl.ds(c * _RC, _RC), :]
        return (z2v + e2c) - ze2c

    def window(c_lo):
        def upd(c, v, M, C):
            msk = v < M
            return jnp.where(msk, v, M), jnp.where(msk, c, C)

        M = jnp.full((_RC, _TBL), jnp.inf, jnp.float32)
        C = jnp.zeros((_RC, _TBL), jnp.int32)
        for i in range(_NCH):
            c = c_lo + i
            M, C = upd(c, chunk_dist(c, chunk_dot(c)), M, C)

        Kg = _RC * C + iota_r
        rc = _RC
        while rc > 1:
            h = rc // 2
            Ma, Mb, Ka, Kb = M[:h], M[h:rc], Kg[:h], Kg[h:rc]
            t = (Ma < Mb) | ((Ma == Mb) & (Ka < Kb))
            M = jnp.where(t, Ma, Mb)
            Kg = jnp.where(t, Ka, Kb)
            rc = h
        return M, Kg

    m0, k0 = window(0)
    m1, k1 = window(_NCH)
    a = m0.astype(jnp.bfloat16).astype(jnp.float32)
    codes_ref[...] = jnp.where(a <= m1, k0, k1).reshape(_TBL)


def _compute_codes(z2r, e2c, zbt, cb2):
    return pl.pallas_call(
        _codes_body,
        grid=(_B // _TBL,),
        in_specs=[
            pl.BlockSpec((1, _TBL), lambda i: (0, i)),
            pl.BlockSpec((_K, 1), lambda i: (0, 0)),
            pl.BlockSpec((_D, _TBL), lambda i: (0, i)),
            pl.BlockSpec((_K, _D), lambda i: (0, 0)),
        ],
        out_specs=pl.BlockSpec((_TBL,), lambda i: (i,)),
        out_shape=jax.ShapeDtypeStruct((_B,), jnp.int32),
    )(z2r, e2c, zbt, cb2)


_NW = 32
_BPW = _B // _NW
_CH = 128


def _gather_body(table_hbm, idx_hbm, out_hbm, idx_v, rows_v, sem):
    wid = lax.axis_index("s") * 2 + lax.axis_index("c")
    base = wid * _BPW
    pltpu.sync_copy(idx_hbm.at[pl.ds(base, _BPW)], idx_v)
    copies = [
        pltpu.async_copy(
            table_hbm.at[idx_v.at[pl.ds(j * _CH, _CH)]],
            rows_v.at[pl.ds(j * _CH, _CH)],
            sem,
        )
        for j in range(_BPW // _CH)
    ]
    for c in copies:
        c.wait()
    pltpu.sync_copy(rows_v, out_hbm.at[pl.ds(base, _BPW)])


@functools.cache
def _gather_rows():
    return pl.kernel(
        _gather_body,
        out_type=jax.ShapeDtypeStruct((_B, _D), jnp.float32),
        mesh=plsc.VectorSubcoreMesh(core_axis_name="c", subcore_axis_name="s"),
        scratch_types=[
            pltpu.VMEM((_BPW,), jnp.int32),
            pltpu.VMEM((_BPW, _D), jnp.float32),
            pltpu.SemaphoreType.DMA,
        ],
        compiler_params=pltpu.CompilerParams(use_tc_tiling_on_sc=False),
    )


def kernel(z_e, codebook):
    z2r = jnp.sum(z_e ** 2, axis=1)[None, :]
    e2c = jnp.sum(codebook ** 2, axis=1, keepdims=True)
    zbt = z_e.T.astype(jnp.bfloat16)
    cb2 = (2.0 * codebook).astype(jnp.bfloat16)
    codes = _compute_codes(z2r, e2c, zbt, cb2)
    z_q = _gather_rows()(codebook, codes)
    return (z_q, codes)
